# baseline (speedup 1.0000x reference)
import dataclasses
import functools

import jax
import jax.numpy as jnp
from jax import lax
from jax.experimental import pallas as pl
from jax.experimental.pallas import tpu as pltpu
from jax.experimental.pallas import tpu_sc as plsc

_BATCH = 4096
_TIME = 200
_DIM = 32
_N = _BATCH * _TIME
_NQ = _N // 4
_NC = 2
_NS = 16
_NW = _NC * _NS
_QPW = _NQ // _NW
_BPW = _N // _NW
_C4 = 320
_STEPS = _QPW // _C4

_TC_B = 32
_TC_G = _TC_B * _TIME // 4
_TC_BLKS = _BATCH // _TC_B


def _compiler_params():
    cp = pltpu.CompilerParams()
    if "use_tc_tiling_on_sc" in pltpu.CompilerParams.__dataclass_fields__:
        cp = dataclasses.replace(cp, use_tc_tiling_on_sc=False)
    return cp


def _sc_gather(W, idxs):
    mesh = plsc.VectorSubcoreMesh(core_axis_name="c", subcore_axis_name="s")

    @functools.partial(
        pl.kernel,
        out_type=jax.ShapeDtypeStruct((_NQ, 128), jnp.float32),
        mesh=mesh,
        scratch_types=[
            pltpu.VMEM((_BPW,), jnp.int32),
            [pltpu.VMEM((_C4, _DIM), jnp.float32) for _ in range(4)],
            [pltpu.VMEM((_C4, _DIM), jnp.float32) for _ in range(4)],
            pltpu.SemaphoreType.DMA,
            pltpu.SemaphoreType.DMA,
            pltpu.SemaphoreType.DMA,
            pltpu.SemaphoreType.DMA,
        ],
        compiler_params=_compiler_params(),
    )
    def gather_kernel(w_hbm, i_hbm, o_hbm, idx_v, rows0, rows1,
                      sg0, sg1, so0, so1):
        wid = lax.axis_index("s") * _NC + lax.axis_index("c")
        base = wid * _QPW

        for k in range(4):
            pltpu.sync_copy(
                i_hbm.at[pl.ds(k * _NQ + base, _QPW)],
                idx_v.at[pl.ds(k * _QPW, _QPW)],
            )

        def out_slice(c, k):
            return o_hbm.at[pl.ds(c, _C4), pl.ds(k * _DIM, _DIM)]

        def drain_store(rows, sem):
            for k in range(4):
                pltpu.make_async_copy(rows[k], out_slice(base, k), sem).wait()

        def gather_chunk(g, rows, sg):
            return [
                pltpu.async_copy(
                    w_hbm.at[idx_v.at[pl.ds(k * _QPW + g * _C4, _C4)]],
                    rows[k], sg,
                )
                for k in range(4)
            ]

        def store_chunk(c, rows, so):
            for k in range(4):
                pltpu.async_copy(rows[k], out_slice(c, k), so)

        @pl.loop(0, _STEPS, step=2)
        def _(g):
            c0 = base + g * _C4
            c1 = c0 + _C4

            @pl.when(g >= 2)
            def _():
                drain_store(rows0, so0)

            h0 = gather_chunk(g, rows0, sg0)

            @pl.when(g >= 2)
            def _():
                drain_store(rows1, so1)

            h1 = gather_chunk(g + 1, rows1, sg1)

            for h in h0:
                h.wait()
            store_chunk(c0, rows0, so0)
            for h in h1:
                h.wait()
            store_chunk(c1, rows1, so1)

        drain_store(rows0, so0)
        drain_store(rows1, so1)

    return gather_kernel(W, idxs)


def _tc_unpack_kernel(g_ref, out_ref):
    g = g_ref[...]
    pieces = [
        g[:, k * _DIM:(k + 1) * _DIM].reshape(_TC_B, _TIME // 4, 1, _DIM)
        for k in range(4)
    ]
    y = jnp.concatenate(pieces, axis=2)
    out_ref[...] = y.reshape(_TC_B, _TIME, _DIM)


def _tc_unpack(G):
    return pl.pallas_call(
        _tc_unpack_kernel,
        grid=(_TC_BLKS,),
        in_specs=[pl.BlockSpec((_TC_G, 128), lambda i: (i, 0))],
        out_specs=pl.BlockSpec((_TC_B, _TIME, _DIM), lambda i: (i, 0, 0)),
        out_shape=jax.ShapeDtypeStruct((_BATCH, _TIME, _DIM), jnp.float32),
        compiler_params=pltpu.CompilerParams(
            dimension_semantics=("arbitrary",),
        ),
    )(G)


def kernel(X, W):
    idx = X.reshape(_N)
    idxs = idx.reshape(_NQ, 4).T.reshape(_N)
    G = _sc_gather(W, idxs)
    return _tc_unpack(G)


# speedup vs baseline: 1.6934x; 1.6934x over previous
import dataclasses
import functools

import jax
import jax.numpy as jnp
from jax import lax
from jax.experimental import pallas as pl
from jax.experimental.pallas import tpu as pltpu
from jax.experimental.pallas import tpu_sc as plsc

_BATCH = 4096
_TIME = 200
_DIM = 32
_N = _BATCH * _TIME
_NH = _N // 2
_NQ = _NH // 4
_NC = 2
_NS = 16
_NW = _NC * _NS
_QPW = _NQ // _NW
_BPW = _NH // _NW
_C4 = 320
_STEPS = _QPW // _C4

_TC_ROWS = 2048
_TC_BLKS = _NQ // _TC_ROWS


def _compiler_params():
    cp = pltpu.CompilerParams()
    if "use_tc_tiling_on_sc" in pltpu.CompilerParams.__dataclass_fields__:
        cp = dataclasses.replace(cp, use_tc_tiling_on_sc=False)
    return cp


def _sc_gather(W, idxs):
    mesh = plsc.VectorSubcoreMesh(core_axis_name="c", subcore_axis_name="s")

    @functools.partial(
        pl.kernel,
        out_type=jax.ShapeDtypeStruct((_NQ, 128), jnp.float32),
        mesh=mesh,
        scratch_types=[
            pltpu.VMEM((_BPW,), jnp.int32),
            [pltpu.VMEM((_C4, _DIM), jnp.float32) for _ in range(4)],
            [pltpu.VMEM((_C4, _DIM), jnp.float32) for _ in range(4)],
            pltpu.SemaphoreType.DMA,
            pltpu.SemaphoreType.DMA,
            pltpu.SemaphoreType.DMA,
            pltpu.SemaphoreType.DMA,
        ],
        compiler_params=_compiler_params(),
    )
    def gather_kernel(w_hbm, i_hbm, o_hbm, idx_v, rows0, rows1,
                      sg0, sg1, so0, so1):
        wid = lax.axis_index("s") * _NC + lax.axis_index("c")
        base = wid * _QPW

        for k in range(4):
            pltpu.sync_copy(
                i_hbm.at[pl.ds(k * _NQ + base, _QPW)],
                idx_v.at[pl.ds(k * _QPW, _QPW)],
            )

        def out_slice(c, k):
            return o_hbm.at[pl.ds(c, _C4), pl.ds(k * _DIM, _DIM)]

        def drain_store(rows, sem):
            for k in range(4):
                pltpu.make_async_copy(rows[k], out_slice(base, k), sem).wait()

        def gather_chunk(g, rows, sg):
            return [
                pltpu.async_copy(
                    w_hbm.at[idx_v.at[pl.ds(k * _QPW + g * _C4, _C4)]],
                    rows[k], sg,
                )
                for k in range(4)
            ]

        def store_chunk(c, rows, so):
            for k in range(4):
                pltpu.async_copy(rows[k], out_slice(c, k), so)

        @pl.loop(0, _STEPS, step=2)
        def _(g):
            c0 = base + g * _C4
            c1 = c0 + _C4

            @pl.when(g >= 2)
            def _():
                drain_store(rows0, so0)

            h0 = gather_chunk(g, rows0, sg0)

            @pl.when(g >= 2)
            def _():
                drain_store(rows1, so1)

            h1 = gather_chunk(g + 1, rows1, sg1)

            for h in h0:
                h.wait()
            store_chunk(c0, rows0, so0)
            for h in h1:
                h.wait()
            store_chunk(c1, rows1, so1)

        drain_store(rows0, so0)
        drain_store(rows1, so1)

    return gather_kernel(W, idxs)


def _tc_unpack_kernel(g1_ref, g2_ref, out_ref):
    g1 = g1_ref[...]
    g2 = g2_ref[...]
    for k in range(4):
        out_ref[0, k, :, :] = g1[:, k * _DIM:(k + 1) * _DIM]
        out_ref[1, k, :, :] = g2[:, k * _DIM:(k + 1) * _DIM]


def _tc_unpack(G1, G2):
    return pl.pallas_call(
        _tc_unpack_kernel,
        grid=(_TC_BLKS,),
        in_specs=[
            pl.BlockSpec((_TC_ROWS, 128), lambda i: (i, 0)),
            pl.BlockSpec((_TC_ROWS, 128), lambda i: (i, 0)),
        ],
        out_specs=pl.BlockSpec(
            (2, 4, _TC_ROWS, _DIM), lambda i: (0, 0, i, 0)
        ),
        out_shape=jax.ShapeDtypeStruct((2, 4, _NQ, _DIM), jnp.float32),
    )(G1, G2)


def kernel(X, W):
    idx = X.reshape(_N)
    G1 = _sc_gather(W, idx[:_NH])
    G2 = _sc_gather(W, idx[_NH:])
    out = _tc_unpack(G1, G2)
    return out.reshape(_BATCH, _TIME, _DIM)
